# Initial kernel scaffold; baseline (speedup 1.0000x reference)
import jax
import jax.numpy as jnp
from jax.experimental import pallas as pl


def kernel(tokens, vocab_table):
    raise NotImplementedError("write your pallas kernel here")


# speedup vs baseline: 139.5056x; 139.5056x over previous
import functools

import jax
import jax.numpy as jnp
from jax import lax
from jax.experimental import pallas as pl
from jax.experimental.pallas import tpu as pltpu
from jax.experimental.pallas import tpu_sc as plsc

BATCH = 16384
HIST = 200
N = BATCH * HIST
NUM_WORKERS = 32
BPW = N // NUM_WORKERS
CHUNK = 25_600
NCHUNK = BPW // CHUNK


def _make_kernel():
    mesh = plsc.VectorSubcoreMesh(core_axis_name="c", subcore_axis_name="s")

    @functools.partial(
        pl.kernel,
        mesh=mesh,
        out_type=jax.ShapeDtypeStruct((N,), jnp.float32),
        scratch_types=[
            pltpu.VMEM((CHUNK,), jnp.int32),
            pltpu.VMEM((CHUNK,), jnp.float32),
            pltpu.SemaphoreType.DMA,
        ],
    )
    def gather_kernel(tok_hbm, tab_hbm, out_hbm, idx_v, val_v, sem):
        wid = lax.axis_index("s") * 2 + lax.axis_index("c")
        base = wid * BPW
        for i in range(NCHUNK):
            off = base + i * CHUNK
            pltpu.sync_copy(tok_hbm.at[pl.ds(off, CHUNK)], idx_v)
            pltpu.async_copy(tab_hbm.at[idx_v], val_v, sem).wait()
            pltpu.sync_copy(val_v, out_hbm.at[pl.ds(off, CHUNK)])

    return gather_kernel


_GATHER = _make_kernel()


def kernel(tokens, vocab_table):
    flat = tokens.reshape(N)
    out = _GATHER(flat, vocab_table)
    return out.reshape(BATCH, HIST)


# speedup vs baseline: 215.3768x; 1.5439x over previous
import functools

import jax
import jax.numpy as jnp
from jax import lax
from jax.experimental import pallas as pl
from jax.experimental.pallas import tpu as pltpu
from jax.experimental.pallas import tpu_sc as plsc

BATCH = 16384
HIST = 200
N = BATCH * HIST
VOCAB_N = 1_000_000
NUM_WORKERS = 32
BPW = N // NUM_WORKERS
CHUNK = 25_600
NCHUNK = BPW // CHUNK
SEG = 25_000
NSEG = VOCAB_N // SEG


def _make_kernel():
    mesh = plsc.VectorSubcoreMesh(core_axis_name="c", subcore_axis_name="s")

    @functools.partial(
        pl.kernel,
        mesh=mesh,
        out_type=jax.ShapeDtypeStruct((N,), jnp.float32),
        scratch_types=[
            pltpu.VMEM_SHARED((VOCAB_N,), jnp.float32),
            pltpu.VMEM((CHUNK,), jnp.int32),
            pltpu.VMEM((CHUNK,), jnp.float32),
            pltpu.SemaphoreType.DMA,
        ],
    )
    def gather_kernel(tok_hbm, tab_hbm, out_hbm, tab_sp, idx_v, val_v, sem):
        s = lax.axis_index("s")
        wid = s * 2 + lax.axis_index("c")
        base = wid * BPW

        for r in range((NSEG + 15) // 16):

            @pl.when(r * 16 + s < NSEG)
            def _():
                toff = (r * 16 + s) * SEG
                pltpu.sync_copy(tab_hbm.at[pl.ds(toff, SEG)],
                                val_v.at[pl.ds(0, SEG)])
                pltpu.sync_copy(val_v.at[pl.ds(0, SEG)],
                                tab_sp.at[pl.ds(toff, SEG)])

        plsc.subcore_barrier()

        for i in range(NCHUNK):
            off = base + i * CHUNK
            pltpu.sync_copy(tok_hbm.at[pl.ds(off, CHUNK)], idx_v)
            pltpu.async_copy(tab_sp.at[idx_v], val_v, sem).wait()
            pltpu.sync_copy(val_v, out_hbm.at[pl.ds(off, CHUNK)])

    return gather_kernel


_GATHER = _make_kernel()


def kernel(tokens, vocab_table):
    flat = tokens.reshape(N)
    out = _GATHER(flat, vocab_table)
    return out.reshape(BATCH, HIST)


# speedup vs baseline: 224.7802x; 1.0437x over previous
import functools

import jax
import jax.numpy as jnp
from jax import lax
from jax.experimental import pallas as pl
from jax.experimental.pallas import tpu as pltpu
from jax.experimental.pallas import tpu_sc as plsc

BATCH = 16384
HIST = 200
N = BATCH * HIST
VOCAB_N = 1_000_000
NUM_WORKERS = 32
BPW = N // NUM_WORKERS
CHUNK = 12_800
NCHUNK = BPW // CHUNK
SEG = 10_000
NSEG = VOCAB_N // SEG


def _make_kernel():
    mesh = plsc.VectorSubcoreMesh(core_axis_name="c", subcore_axis_name="s")

    @functools.partial(
        pl.kernel,
        mesh=mesh,
        out_type=jax.ShapeDtypeStruct((N,), jnp.float32),
        scratch_types=[
            pltpu.VMEM_SHARED((VOCAB_N,), jnp.float32),
            pltpu.VMEM((CHUNK,), jnp.int32),
            pltpu.VMEM((CHUNK,), jnp.int32),
            pltpu.VMEM((CHUNK,), jnp.float32),
            pltpu.VMEM((CHUNK,), jnp.float32),
            pltpu.SemaphoreType.DMA,
            pltpu.SemaphoreType.DMA,
            pltpu.SemaphoreType.DMA,
            pltpu.SemaphoreType.DMA,
            pltpu.SemaphoreType.DMA,
        ],
    )
    def gather_kernel(tok_hbm, tab_hbm, out_hbm, tab_sp, idx0, idx1,
                      val0, val1, si0, si1, so0, so1, sg):
        s = lax.axis_index("s")
        wid = s * 2 + lax.axis_index("c")
        base = wid * BPW
        idx = (idx0, idx1)
        val = (val0, val1)
        sem_i = (si0, si1)
        sem_o = (so0, so1)

        for r in range((NSEG + 15) // 16):

            @pl.when(r * 16 + s < NSEG)
            def _():
                toff = (r * 16 + s) * SEG
                pltpu.sync_copy(tab_hbm.at[pl.ds(toff, SEG)],
                                val0.at[pl.ds(0, SEG)])
                pltpu.sync_copy(val0.at[pl.ds(0, SEG)],
                                tab_sp.at[pl.ds(toff, SEG)])

        plsc.subcore_barrier()

        pltpu.async_copy(tok_hbm.at[pl.ds(base, CHUNK)], idx0, si0)
        for i in range(NCHUNK):
            b = i % 2
            if i + 1 < NCHUNK:
                pltpu.async_copy(
                    tok_hbm.at[pl.ds(base + (i + 1) * CHUNK, CHUNK)],
                    idx[1 - b], sem_i[1 - b])
            if i >= 2:
                pltpu.make_async_copy(
                    val[b], out_hbm.at[pl.ds(base + (i - 2) * CHUNK, CHUNK)],
                    sem_o[b]).wait()
            pltpu.make_async_copy(
                tok_hbm.at[pl.ds(base + i * CHUNK, CHUNK)], idx[b],
                sem_i[b]).wait()
            pltpu.async_copy(tab_sp.at[idx[b]], val[b], sg).wait()
            pltpu.async_copy(
                val[b], out_hbm.at[pl.ds(base + i * CHUNK, CHUNK)], sem_o[b])
        for i in range(NCHUNK - 2, NCHUNK):
            b = i % 2
            pltpu.make_async_copy(
                val[b], out_hbm.at[pl.ds(base + i * CHUNK, CHUNK)],
                sem_o[b]).wait()

    return gather_kernel


_GATHER = _make_kernel()


def kernel(tokens, vocab_table):
    flat = tokens.reshape(N)
    out = _GATHER(flat, vocab_table)
    return out.reshape(BATCH, HIST)
